# baseline (speedup 1.0000x reference)
import functools

import jax
import jax.numpy as jnp
import numpy as np
from jax.experimental import pallas as pl

_DIM = 768
_EMB = 64
_K = 1024
_B = 128
_SEQ = 256
_EPS = 1e-12
_BB = 2
_RB = 256
_N = _B * 16

_HIGH = jax.lax.Precision.DEFAULT


def _tap_matrix(in_hw: int, out_hw: int) -> np.ndarray:
    P = out_hw * out_hw
    M = np.zeros((9 * P, in_hw * in_hw), np.float32)
    for kh in range(3):
        for kw in range(3):
            t = kh * 3 + kw
            for oh in range(out_hw):
                for ow in range(out_hw):
                    ih, iw = 2 * oh + kh - 1, 2 * ow + kw - 1
                    if 0 <= ih < in_hw and 0 <= iw < in_hw:
                        M[t * P + oh * out_hw + ow, ih * in_hw + iw] = 1.0
    return M


_P1 = _tap_matrix(16, 8)
_P2 = _tap_matrix(8, 4)


def _dot(a, b):
    return jax.lax.dot_general(a, b, (((1,), (0,)), ((), ())),
                               precision=_HIGH,
                               preferred_element_type=jnp.float32)


def _encode_one(a, p1_ref, w1_ref, b1, p2_ref, w2_ref):
    patches = _dot(p1_ref[...], a)
    h = _dot(patches[0:64], w1_ref[0])
    for t in range(1, 9):
        h = h + _dot(patches[64 * t:64 * (t + 1)], w1_ref[t])
    return jax.nn.relu(h + b1)


def _conv2(d, p2_ref, w2_ref):
    patches = _dot(p2_ref[...], d)
    x = _dot(patches[0:16], w2_ref[0])
    for t in range(1, 9):
        x = x + _dot(patches[16 * t:16 * (t + 1)], w2_ref[t])
    return x


def _kernel_a(first_ref, last_ref, win_ref, bin_ref, cbt_ref, w1_ref, b1_ref,
              w2_ref, b2_ref, p1_ref, p2_ref, rv_ref, q_ref, idx_ref, acc_ref):
    step = pl.program_id(0)

    @pl.when(step == 0)
    def _():
        acc_ref[...] = jnp.zeros_like(acc_ref)

    win = win_ref[...]
    b_in = bin_ref[...]
    b1 = b1_ref[...]
    cbt = cbt_ref[...]
    cb_sq = jnp.sum(cbt * cbt, axis=0, keepdims=True)
    lane = jax.lax.broadcasted_iota(jnp.int32, (16, _K), 1).astype(jnp.float32)

    a_f = _dot(first_ref[...].reshape(_BB * _SEQ, _DIM), win) + b_in
    a_l = _dot(last_ref[...].reshape(_BB * _SEQ, _DIM), win) + b_in

    acc = acc_ref[...]
    for i in range(_BB):
        r_f = _encode_one(a_f[i * _SEQ:(i + 1) * _SEQ], p1_ref, w1_ref, b1,
                          p2_ref, w2_ref)
        r_l = _encode_one(a_l[i * _SEQ:(i + 1) * _SEQ], p1_ref, w1_ref, b1,
                          p2_ref, w2_ref)
        b2 = b2_ref[...]
        x = ((_conv2(r_l, p2_ref, w2_ref) + b2)
             - (_conv2(r_f, p2_ref, w2_ref) + b2))

        x_sq = jnp.sum(x * x, axis=1, keepdims=True)
        dist = x_sq - 2.0 * _dot(x, cbt) + cb_sq
        md = jnp.min(dist, axis=1, keepdims=True)
        hit = dist <= md
        idxf = jnp.min(jnp.where(hit, lane, float(_K)), axis=1,
                       keepdims=True)

        n_res = jnp.sqrt(jnp.maximum(md, 0.0))
        rv = rv_ref[i]
        n_rv = jnp.sqrt(jnp.sum(rv * rv, axis=1, keepdims=True))
        q_ref[i] = x + (n_res / n_rv + _EPS) * rv
        idx_ref[i] = idxf
        acc = acc + jnp.where(lane == idxf, 1.0, 0.0)
    acc_ref[...] = acc


def _kernel_b(qs_ref, wout_ref, bout_ref, acc_ref, cu_ref,
              out_ref, perp_ref, used_ref):
    @pl.when(pl.program_id(0) == 0)
    def _():
        counts = jnp.sum(acc_ref[...], axis=0, keepdims=True)
        used_ref[...] = cu_ref[...] + counts
        avg = counts * (1.0 / _N)
        ent = jnp.sum(avg * jnp.log(avg + _EPS), axis=1, keepdims=True)
        perp_ref[...] = jnp.exp(-ent)

    out_ref[...] = _dot(qs_ref[...], wout_ref[...]) + bout_ref[...]


def kernel(input_data_first, input_data_last, codebooks, W_in, b_in,
           conv1_w, conv1_b, conv2_w, conv2_b, W_out, b_out, codebooks_used):
    bsz = input_data_first.shape[0]
    cbt = codebooks.T
    w1 = conv1_w.transpose(2, 3, 1, 0).reshape(9, _EMB, _EMB)
    w2 = conv2_w.transpose(2, 3, 1, 0).reshape(9, _EMB, _EMB)
    rv = jax.random.normal(jax.random.key(42), (_N, _EMB),
                           jnp.float32).reshape(bsz, 16, _EMB)
    p1 = jnp.asarray(_P1)
    p2 = jnp.asarray(_P2)

    steps = bsz // _BB
    q, idxf, acc = pl.pallas_call(
        _kernel_a,
        grid=(steps,),
        in_specs=[
            pl.BlockSpec((_BB, _SEQ, _DIM), lambda s: (s, 0, 0)),
            pl.BlockSpec((_BB, _SEQ, _DIM), lambda s: (s, 0, 0)),
            pl.BlockSpec((_DIM, _EMB), lambda s: (0, 0)),
            pl.BlockSpec((1, _EMB), lambda s: (0, 0)),
            pl.BlockSpec((_EMB, _K), lambda s: (0, 0)),
            pl.BlockSpec((9, _EMB, _EMB), lambda s: (0, 0, 0)),
            pl.BlockSpec((1, _EMB), lambda s: (0, 0)),
            pl.BlockSpec((9, _EMB, _EMB), lambda s: (0, 0, 0)),
            pl.BlockSpec((1, _EMB), lambda s: (0, 0)),
            pl.BlockSpec((576, _SEQ), lambda s: (0, 0)),
            pl.BlockSpec((144, _EMB), lambda s: (0, 0)),
            pl.BlockSpec((_BB, 16, _EMB), lambda s: (s, 0, 0)),
        ],
        out_specs=[
            pl.BlockSpec((_BB, 16, _EMB), lambda s: (s, 0, 0)),
            pl.BlockSpec((_BB, 16, 1), lambda s: (s, 0, 0)),
            pl.BlockSpec((16, _K), lambda s: (0, 0)),
        ],
        out_shape=[
            jax.ShapeDtypeStruct((bsz, 16, _EMB), jnp.float32),
            jax.ShapeDtypeStruct((bsz, 16, 1), jnp.float32),
            jax.ShapeDtypeStruct((16, _K), jnp.float32),
        ],
    )(input_data_first, input_data_last, W_in, b_in.reshape(1, _EMB), cbt,
      w1, conv1_b.reshape(1, _EMB), w2, conv2_b.reshape(1, _EMB), p1, p2, rv)

    qs = q.reshape(bsz, _EMB, 16).transpose(0, 2, 1).reshape(bsz * 16, _EMB)

    out, perp, used_f = pl.pallas_call(
        _kernel_b,
        grid=(bsz * 16 // _RB,),
        in_specs=[
            pl.BlockSpec((_RB, _EMB), lambda s: (s, 0)),
            pl.BlockSpec((_EMB, _DIM), lambda s: (0, 0)),
            pl.BlockSpec((1, _DIM), lambda s: (0, 0)),
            pl.BlockSpec((16, _K), lambda s: (0, 0)),
            pl.BlockSpec((1, _K), lambda s: (0, 0)),
        ],
        out_specs=[
            pl.BlockSpec((_RB, _DIM), lambda s: (s, 0)),
            pl.BlockSpec((1, 1), lambda s: (0, 0)),
            pl.BlockSpec((1, _K), lambda s: (0, 0)),
        ],
        out_shape=[
            jax.ShapeDtypeStruct((bsz * 16, _DIM), jnp.float32),
            jax.ShapeDtypeStruct((1, 1), jnp.float32),
            jax.ShapeDtypeStruct((1, _K), jnp.float32),
        ],
    )(qs, W_out, b_out.reshape(1, _DIM), acc,
      codebooks_used.astype(jnp.float32).reshape(1, _K))

    out = out.reshape(bsz, 16, _DIM)
    perplexity = perp.reshape(())
    used = used_f.reshape(_K).astype(jnp.int32)
    min_indices = idxf.reshape(bsz, 16).astype(jnp.int32)
    return out, perplexity, used, min_indices


# speedup vs baseline: 1.3380x; 1.3380x over previous
import functools

import jax
import jax.numpy as jnp
import numpy as np
from jax.experimental import pallas as pl

_DIM = 768
_EMB = 64
_K = 1024
_B = 128
_SEQ = 256
_EPS = 1e-12
_BB = 2
_RB = 256
_N = _B * 16

_HIGH = jax.lax.Precision.DEFAULT


def _dot(a, b):
    return jax.lax.dot_general(a, b, (((1,), (0,)), ((), ())),
                               precision=_HIGH,
                               preferred_element_type=jnp.float32)


def _planes(img, out_hw):
    C = img.shape[-1]
    t = img.reshape(out_hw, 2, 2 * out_hw, C)
    E, O = t[:, 0], t[:, 1]
    Os = jnp.concatenate([jnp.zeros_like(O[:1]), O[:-1]], axis=0)
    out = []
    for r in (Os, E, O):
        rc = r.reshape(out_hw, out_hw, 2, C)
        CE, CO = rc[:, :, 0], rc[:, :, 1]
        COs = jnp.concatenate([jnp.zeros_like(CO[:, :1]), CO[:, :-1]], axis=1)
        for c in (COs, CE, CO):
            out.append(c.reshape(out_hw * out_hw, C))
    return out


def _encode_one(a, w1_ref, b1):
    p = _planes(a, 8)
    h = _dot(p[0], w1_ref[0])
    for t in range(1, 9):
        h = h + _dot(p[t], w1_ref[t])
    return jax.nn.relu(h + b1)


def _conv2(d, w2_ref):
    p = _planes(d, 4)
    x = _dot(p[0], w2_ref[0])
    for t in range(1, 9):
        x = x + _dot(p[t], w2_ref[t])
    return x


def _kernel_a(first_ref, last_ref, win_ref, bin_ref, cbt_ref, w1_ref, b1_ref,
              w2_ref, b2_ref, rv_ref, q_ref, idx_ref, acc_ref):
    step = pl.program_id(0)

    @pl.when(step == 0)
    def _():
        acc_ref[...] = jnp.zeros_like(acc_ref)

    win = win_ref[...]
    b_in = bin_ref[...]
    b1 = b1_ref[...]
    b2 = b2_ref[...]
    cbt = cbt_ref[...]
    cb_sq = jnp.sum(cbt * cbt, axis=0, keepdims=True)
    nr = _BB * 16
    lane = jax.lax.broadcasted_iota(jnp.int32, (nr, _K), 1).astype(jnp.float32)

    a_f = _dot(first_ref[...].reshape(_BB * _SEQ, _DIM), win) + b_in
    a_l = _dot(last_ref[...].reshape(_BB * _SEQ, _DIM), win) + b_in

    xs = []
    for i in range(_BB):
        r_f = _encode_one(a_f[i * _SEQ:(i + 1) * _SEQ], w1_ref, b1)
        r_l = _encode_one(a_l[i * _SEQ:(i + 1) * _SEQ], w1_ref, b1)
        xs.append((_conv2(r_l, w2_ref) + b2) - (_conv2(r_f, w2_ref) + b2))
    x = jnp.concatenate(xs, axis=0)

    x_sq = jnp.sum(x * x, axis=1, keepdims=True)
    dist = x_sq - 2.0 * _dot(x, cbt) + cb_sq
    md = jnp.min(dist, axis=1, keepdims=True)
    idxf = jnp.min(jnp.where(dist <= md, lane, float(_K)), axis=1,
                   keepdims=True)

    n_res = jnp.sqrt(jnp.maximum(md, 0.0))
    rv = rv_ref[...].reshape(nr, _EMB)
    n_rv = jnp.sqrt(jnp.sum(rv * rv, axis=1, keepdims=True))
    q_ref[...] = (x + (n_res / n_rv + _EPS) * rv).reshape(_BB, 16, _EMB)
    idx_ref[...] = idxf.reshape(_BB, 16, 1)

    onehot = jnp.where(lane == idxf, 1.0, 0.0)
    acc = acc_ref[...]
    for i in range(_BB):
        acc = acc + onehot[i * 16:(i + 1) * 16]
    acc_ref[...] = acc


def _kernel_b(qs_ref, wout_ref, bout_ref, acc_ref, cu_ref,
              out_ref, perp_ref, used_ref):
    @pl.when(pl.program_id(0) == 0)
    def _():
        counts = jnp.sum(acc_ref[...], axis=0, keepdims=True)
        used_ref[...] = cu_ref[...] + counts
        avg = counts * (1.0 / _N)
        ent = jnp.sum(avg * jnp.log(avg + _EPS), axis=1, keepdims=True)
        perp_ref[...] = jnp.exp(-ent)

    out_ref[...] = _dot(qs_ref[...], wout_ref[...]) + bout_ref[...]


def kernel(input_data_first, input_data_last, codebooks, W_in, b_in,
           conv1_w, conv1_b, conv2_w, conv2_b, W_out, b_out, codebooks_used):
    bsz = input_data_first.shape[0]
    cbt = codebooks.T
    w1 = conv1_w.transpose(2, 3, 1, 0).reshape(9, _EMB, _EMB)
    w2 = conv2_w.transpose(2, 3, 1, 0).reshape(9, _EMB, _EMB)
    rv = jax.random.normal(jax.random.key(42), (_N, _EMB),
                           jnp.float32).reshape(bsz, 16, _EMB)
    steps = bsz // _BB
    q, idxf, acc = pl.pallas_call(
        _kernel_a,
        grid=(steps,),
        in_specs=[
            pl.BlockSpec((_BB, _SEQ, _DIM), lambda s: (s, 0, 0)),
            pl.BlockSpec((_BB, _SEQ, _DIM), lambda s: (s, 0, 0)),
            pl.BlockSpec((_DIM, _EMB), lambda s: (0, 0)),
            pl.BlockSpec((1, _EMB), lambda s: (0, 0)),
            pl.BlockSpec((_EMB, _K), lambda s: (0, 0)),
            pl.BlockSpec((9, _EMB, _EMB), lambda s: (0, 0, 0)),
            pl.BlockSpec((1, _EMB), lambda s: (0, 0)),
            pl.BlockSpec((9, _EMB, _EMB), lambda s: (0, 0, 0)),
            pl.BlockSpec((1, _EMB), lambda s: (0, 0)),
            pl.BlockSpec((_BB, 16, _EMB), lambda s: (s, 0, 0)),
        ],
        out_specs=[
            pl.BlockSpec((_BB, 16, _EMB), lambda s: (s, 0, 0)),
            pl.BlockSpec((_BB, 16, 1), lambda s: (s, 0, 0)),
            pl.BlockSpec((16, _K), lambda s: (0, 0)),
        ],
        out_shape=[
            jax.ShapeDtypeStruct((bsz, 16, _EMB), jnp.float32),
            jax.ShapeDtypeStruct((bsz, 16, 1), jnp.float32),
            jax.ShapeDtypeStruct((16, _K), jnp.float32),
        ],
    )(input_data_first, input_data_last, W_in, b_in.reshape(1, _EMB), cbt,
      w1, conv1_b.reshape(1, _EMB), w2, conv2_b.reshape(1, _EMB), rv)

    qs = q.reshape(bsz, _EMB, 16).transpose(0, 2, 1).reshape(bsz * 16, _EMB)

    out, perp, used_f = pl.pallas_call(
        _kernel_b,
        grid=(bsz * 16 // _RB,),
        in_specs=[
            pl.BlockSpec((_RB, _EMB), lambda s: (s, 0)),
            pl.BlockSpec((_EMB, _DIM), lambda s: (0, 0)),
            pl.BlockSpec((1, _DIM), lambda s: (0, 0)),
            pl.BlockSpec((16, _K), lambda s: (0, 0)),
            pl.BlockSpec((1, _K), lambda s: (0, 0)),
        ],
        out_specs=[
            pl.BlockSpec((_RB, _DIM), lambda s: (s, 0)),
            pl.BlockSpec((1, 1), lambda s: (0, 0)),
            pl.BlockSpec((1, _K), lambda s: (0, 0)),
        ],
        out_shape=[
            jax.ShapeDtypeStruct((bsz * 16, _DIM), jnp.float32),
            jax.ShapeDtypeStruct((1, 1), jnp.float32),
            jax.ShapeDtypeStruct((1, _K), jnp.float32),
        ],
    )(qs, W_out, b_out.reshape(1, _DIM), acc,
      codebooks_used.astype(jnp.float32).reshape(1, _K))

    out = out.reshape(bsz, 16, _DIM)
    perplexity = perp.reshape(())
    used = used_f.reshape(_K).astype(jnp.int32)
    min_indices = idxf.reshape(bsz, 16).astype(jnp.int32)
    return out, perplexity, used, min_indices


# speedup vs baseline: 1.4672x; 1.0966x over previous
import functools

import jax
import jax.numpy as jnp
import numpy as np
from jax.experimental import pallas as pl
from jax.experimental.pallas import tpu as pltpu

_DIM = 768
_EMB = 64
_K = 1024
_B = 128
_SEQ = 256
_EPS = 1e-12
_BB = 2
_RB = 256
_N = _B * 16

_HIGH = jax.lax.Precision.DEFAULT


def _dot(a, b):
    return jax.lax.dot_general(a, b, (((1,), (0,)), ((), ())),
                               precision=_HIGH,
                               preferred_element_type=jnp.float32)


def _tap_views(scr, j, ohw, m):
    p = {}
    for hp in (0, 1):
        for wp in (0, 1):
            v = scr[pl.ds(j, 1), pl.Slice(hp, ohw, 2), pl.Slice(wp, ohw, 2), :]
            p[(hp, wp)] = v.reshape(ohw * ohw, _EMB)
    z = jnp.zeros((ohw, _EMB), jnp.float32)
    z1 = jnp.zeros((1, _EMB), jnp.float32)

    def sh(x):
        return jnp.concatenate([z, x[:-ohw]], axis=0)

    def sw(x):
        return jnp.concatenate([z1, x[:-1]], axis=0) * m

    views = []
    for hp, shf in ((1, True), (0, False), (1, False)):
        for wp, swf in ((1, True), (0, False), (1, False)):
            v = p[(hp, wp)]
            if swf:
                v = sw(v)
            if shf:
                v = sh(v)
            views.append(v)
    return views


def _conv(scr, j, ohw, m, w_ref):
    vs = _tap_views(scr, j, ohw, m)
    h = _dot(vs[0], w_ref[0])
    for t in range(1, 9):
        h = h + _dot(vs[t], w_ref[t])
    return h


def _kernel_a(first_ref, last_ref, win_ref, bin_ref, cbt_ref, w1_ref, b1_ref,
              w2_ref, b2_ref, rv_ref, q_ref, idx_ref, acc_ref,
              scr1, scr2):
    step = pl.program_id(0)

    @pl.when(step == 0)
    def _():
        acc_ref[...] = jnp.zeros_like(acc_ref)

    win = win_ref[...]
    b_in = bin_ref[...]
    b1 = b1_ref[...]
    b2 = b2_ref[...]
    cbt = cbt_ref[...]
    cb_sq = jnp.sum(cbt * cbt, axis=0, keepdims=True)
    nr = _BB * 16
    lane = jax.lax.broadcasted_iota(jnp.int32, (nr, _K), 1).astype(jnp.float32)

    a_f = _dot(first_ref[...].reshape(_BB * _SEQ, _DIM), win) + b_in
    a_l = _dot(last_ref[...].reshape(_BB * _SEQ, _DIM), win) + b_in

    m1 = (jax.lax.broadcasted_iota(jnp.int32, (64, 1), 0) % 8
          != 0).astype(jnp.float32)
    m2 = (jax.lax.broadcasted_iota(jnp.int32, (16, 1), 0) % 4
          != 0).astype(jnp.float32)
    for i in range(_BB):
        scr1[i] = a_f[i * _SEQ:(i + 1) * _SEQ].reshape(16, 16, _EMB)
        scr1[_BB + i] = a_l[i * _SEQ:(i + 1) * _SEQ].reshape(16, 16, _EMB)
    for j in range(2 * _BB):
        r = jax.nn.relu(_conv(scr1, j, 8, m1, w1_ref) + b1)
        scr2[j] = r.reshape(8, 8, _EMB)
    xs = []
    for i in range(_BB):
        xs.append((_conv(scr2, _BB + i, 4, m2, w2_ref) + b2)
                  - (_conv(scr2, i, 4, m2, w2_ref) + b2))
    x = jnp.concatenate(xs, axis=0)

    x_sq = jnp.sum(x * x, axis=1, keepdims=True)
    dist = x_sq - 2.0 * _dot(x, cbt) + cb_sq
    md = jnp.min(dist, axis=1, keepdims=True)
    idxf = jnp.min(jnp.where(dist <= md, lane, float(_K)), axis=1,
                   keepdims=True)

    n_res = jnp.sqrt(jnp.maximum(md, 0.0))
    rv = rv_ref[...].reshape(nr, _EMB)
    n_rv = jnp.sqrt(jnp.sum(rv * rv, axis=1, keepdims=True))
    q_ref[...] = (x + (n_res / n_rv + _EPS) * rv).reshape(_BB, 16, _EMB)
    idx_ref[...] = idxf.reshape(_BB, 16, 1)

    onehot = jnp.where(lane == idxf, 1.0, 0.0)
    acc = acc_ref[...]
    for i in range(_BB):
        acc = acc + onehot[i * 16:(i + 1) * 16]
    acc_ref[...] = acc


def _kernel_b(qs_ref, wout_ref, bout_ref, acc_ref, cu_ref,
              out_ref, perp_ref, used_ref):
    @pl.when(pl.program_id(0) == 0)
    def _():
        counts = jnp.sum(acc_ref[...], axis=0, keepdims=True)
        used_ref[...] = cu_ref[...] + counts
        avg = counts * (1.0 / _N)
        ent = jnp.sum(avg * jnp.log(avg + _EPS), axis=1, keepdims=True)
        perp_ref[...] = jnp.exp(-ent)

    out_ref[...] = _dot(qs_ref[...], wout_ref[...]) + bout_ref[...]


def kernel(input_data_first, input_data_last, codebooks, W_in, b_in,
           conv1_w, conv1_b, conv2_w, conv2_b, W_out, b_out, codebooks_used):
    bsz = input_data_first.shape[0]
    cbt = codebooks.T
    w1 = conv1_w.transpose(2, 3, 1, 0).reshape(9, _EMB, _EMB)
    w2 = conv2_w.transpose(2, 3, 1, 0).reshape(9, _EMB, _EMB)
    rv = jax.random.normal(jax.random.key(42), (_N, _EMB),
                           jnp.float32).reshape(bsz, 16, _EMB)
    steps = bsz // _BB
    q, idxf, acc = pl.pallas_call(
        _kernel_a,
        grid=(steps,),
        in_specs=[
            pl.BlockSpec((_BB, _SEQ, _DIM), lambda s: (s, 0, 0)),
            pl.BlockSpec((_BB, _SEQ, _DIM), lambda s: (s, 0, 0)),
            pl.BlockSpec((_DIM, _EMB), lambda s: (0, 0)),
            pl.BlockSpec((1, _EMB), lambda s: (0, 0)),
            pl.BlockSpec((_EMB, _K), lambda s: (0, 0)),
            pl.BlockSpec((9, _EMB, _EMB), lambda s: (0, 0, 0)),
            pl.BlockSpec((1, _EMB), lambda s: (0, 0)),
            pl.BlockSpec((9, _EMB, _EMB), lambda s: (0, 0, 0)),
            pl.BlockSpec((1, _EMB), lambda s: (0, 0)),
            pl.BlockSpec((_BB, 16, _EMB), lambda s: (s, 0, 0)),
        ],
        scratch_shapes=[
            pltpu.VMEM((2 * _BB, 16, 16, _EMB), jnp.float32),
            pltpu.VMEM((2 * _BB, 8, 8, _EMB), jnp.float32),
        ],
        out_specs=[
            pl.BlockSpec((_BB, 16, _EMB), lambda s: (s, 0, 0)),
            pl.BlockSpec((_BB, 16, 1), lambda s: (s, 0, 0)),
            pl.BlockSpec((16, _K), lambda s: (0, 0)),
        ],
        out_shape=[
            jax.ShapeDtypeStruct((bsz, 16, _EMB), jnp.float32),
            jax.ShapeDtypeStruct((bsz, 16, 1), jnp.float32),
            jax.ShapeDtypeStruct((16, _K), jnp.float32),
        ],
    )(input_data_first, input_data_last, W_in, b_in.reshape(1, _EMB), cbt,
      w1, conv1_b.reshape(1, _EMB), w2, conv2_b.reshape(1, _EMB), rv)

    qs = q.reshape(bsz, _EMB, 16).transpose(0, 2, 1).reshape(bsz * 16, _EMB)

    out, perp, used_f = pl.pallas_call(
        _kernel_b,
        grid=(bsz * 16 // _RB,),
        in_specs=[
            pl.BlockSpec((_RB, _EMB), lambda s: (s, 0)),
            pl.BlockSpec((_EMB, _DIM), lambda s: (0, 0)),
            pl.BlockSpec((1, _DIM), lambda s: (0, 0)),
            pl.BlockSpec((16, _K), lambda s: (0, 0)),
            pl.BlockSpec((1, _K), lambda s: (0, 0)),
        ],
        out_specs=[
            pl.BlockSpec((_RB, _DIM), lambda s: (s, 0)),
            pl.BlockSpec((1, 1), lambda s: (0, 0)),
            pl.BlockSpec((1, _K), lambda s: (0, 0)),
        ],
        out_shape=[
            jax.ShapeDtypeStruct((bsz * 16, _DIM), jnp.float32),
            jax.ShapeDtypeStruct((1, 1), jnp.float32),
            jax.ShapeDtypeStruct((1, _K), jnp.float32),
        ],
    )(qs, W_out, b_out.reshape(1, _DIM), acc,
      codebooks_used.astype(jnp.float32).reshape(1, _K))

    out = out.reshape(bsz, 16, _DIM)
    perplexity = perp.reshape(())
    used = used_f.reshape(_K).astype(jnp.int32)
    min_indices = idxf.reshape(bsz, 16).astype(jnp.int32)
    return out, perplexity, used, min_indices


# speedup vs baseline: 1.8389x; 1.2534x over previous
import functools

import jax
import jax.numpy as jnp
import numpy as np
from jax.experimental import pallas as pl
from jax.experimental.pallas import tpu as pltpu

_DIM = 768
_EMB = 64
_K = 1024
_B = 128
_SEQ = 256
_EPS = 1e-12
_BB = 4
_RB = 256
_N = _B * 16

_HIGH = jax.lax.Precision.DEFAULT


def _dot(a, b):
    return jax.lax.dot_general(a, b, (((1,), (0,)), ((), ())),
                               precision=_HIGH,
                               preferred_element_type=jnp.float32)


def _tap_views(scr, j, ohw, m):
    p = {}
    for hp in (0, 1):
        for wp in (0, 1):
            v = scr[pl.ds(j, 1), pl.Slice(hp, ohw, 2), pl.Slice(wp, ohw, 2), :]
            p[(hp, wp)] = v.reshape(ohw * ohw, _EMB)
    z = jnp.zeros((ohw, _EMB), jnp.float32)
    z1 = jnp.zeros((1, _EMB), jnp.float32)

    def sh(x):
        return jnp.concatenate([z, x[:-ohw]], axis=0)

    def sw(x):
        return jnp.concatenate([z1, x[:-1]], axis=0) * m

    views = []
    for hp, shf in ((1, True), (0, False), (1, False)):
        for wp, swf in ((1, True), (0, False), (1, False)):
            v = p[(hp, wp)]
            if swf:
                v = sw(v)
            if shf:
                v = sh(v)
            views.append(v)
    return views


def _conv(scr, j, ohw, m, w_ref):
    vs = _tap_views(scr, j, ohw, m)
    h = _dot(vs[0], w_ref[0])
    for t in range(1, 9):
        h = h + _dot(vs[t], w_ref[t])
    return h


def _kernel_a(first_ref, last_ref, win_ref, bin_ref, cbt_ref, w1_ref, b1_ref,
              w2_ref, b2_ref, rv_ref, q_ref, idx_ref, acc_ref,
              scr1, scr2):
    step = pl.program_id(0)

    @pl.when(step == 0)
    def _():
        acc_ref[...] = jnp.zeros_like(acc_ref)

    win = win_ref[...]
    b_in = bin_ref[...]
    b1 = b1_ref[...]
    b2 = b2_ref[...]
    cbt = cbt_ref[...]
    cb_sq = jnp.sum(cbt * cbt, axis=0, keepdims=True)
    nr = _BB * 16
    lane = jax.lax.broadcasted_iota(jnp.int32, (nr, _K), 1).astype(jnp.float32)

    a_f = _dot(first_ref[...].reshape(_BB * _SEQ, _DIM), win) + b_in
    a_l = _dot(last_ref[...].reshape(_BB * _SEQ, _DIM), win) + b_in

    m1 = (jax.lax.broadcasted_iota(jnp.int32, (64, 1), 0) % 8
          != 0).astype(jnp.float32)
    m2 = (jax.lax.broadcasted_iota(jnp.int32, (16, 1), 0) % 4
          != 0).astype(jnp.float32)
    for i in range(_BB):
        scr1[i] = a_f[i * _SEQ:(i + 1) * _SEQ].reshape(16, 16, _EMB)
        scr1[_BB + i] = a_l[i * _SEQ:(i + 1) * _SEQ].reshape(16, 16, _EMB)
    for j in range(2 * _BB):
        r = jax.nn.relu(_conv(scr1, j, 8, m1, w1_ref) + b1)
        scr2[j] = r.reshape(8, 8, _EMB)
    xs = []
    for i in range(_BB):
        xs.append((_conv(scr2, _BB + i, 4, m2, w2_ref) + b2)
                  - (_conv(scr2, i, 4, m2, w2_ref) + b2))
    x = jnp.concatenate(xs, axis=0)

    x_sq = jnp.sum(x * x, axis=1, keepdims=True)
    dist = x_sq - 2.0 * _dot(x, cbt) + cb_sq
    md = jnp.min(dist, axis=1, keepdims=True)
    idxf = jnp.min(jnp.where(dist <= md, lane, float(_K)), axis=1,
                   keepdims=True)

    n_res = jnp.sqrt(jnp.maximum(md, 0.0))
    rv = rv_ref[...].reshape(nr, _EMB)
    n_rv = jnp.sqrt(jnp.sum(rv * rv, axis=1, keepdims=True))
    q_ref[...] = (x + (n_res / n_rv + _EPS) * rv).reshape(_BB, 16, _EMB)
    idx_ref[...] = idxf.reshape(_BB, 16, 1)

    onehot = jnp.where(lane == idxf, 1.0, 0.0)
    acc = acc_ref[...]
    for i in range(_BB):
        acc = acc + onehot[i * 16:(i + 1) * 16]
    acc_ref[...] = acc


def _kernel_b(qs_ref, wout_ref, bout_ref, acc_ref, cu_ref,
              out_ref, perp_ref, used_ref):
    @pl.when(pl.program_id(0) == 0)
    def _():
        counts = jnp.sum(acc_ref[...], axis=0, keepdims=True)
        used_ref[...] = cu_ref[...] + counts
        avg = counts * (1.0 / _N)
        ent = jnp.sum(avg * jnp.log(avg + _EPS), axis=1, keepdims=True)
        perp_ref[...] = jnp.exp(-ent)

    out_ref[...] = _dot(qs_ref[...], wout_ref[...]) + bout_ref[...]


def kernel(input_data_first, input_data_last, codebooks, W_in, b_in,
           conv1_w, conv1_b, conv2_w, conv2_b, W_out, b_out, codebooks_used):
    bsz = input_data_first.shape[0]
    cbt = codebooks.T
    w1 = conv1_w.transpose(2, 3, 1, 0).reshape(9, _EMB, _EMB)
    w2 = conv2_w.transpose(2, 3, 1, 0).reshape(9, _EMB, _EMB)
    rv = jax.random.normal(jax.random.key(42), (_N, _EMB),
                           jnp.float32).reshape(bsz, 16, _EMB)
    steps = bsz // _BB
    q, idxf, acc = pl.pallas_call(
        _kernel_a,
        grid=(steps,),
        in_specs=[
            pl.BlockSpec((_BB, _SEQ, _DIM), lambda s: (s, 0, 0)),
            pl.BlockSpec((_BB, _SEQ, _DIM), lambda s: (s, 0, 0)),
            pl.BlockSpec((_DIM, _EMB), lambda s: (0, 0)),
            pl.BlockSpec((1, _EMB), lambda s: (0, 0)),
            pl.BlockSpec((_EMB, _K), lambda s: (0, 0)),
            pl.BlockSpec((9, _EMB, _EMB), lambda s: (0, 0, 0)),
            pl.BlockSpec((1, _EMB), lambda s: (0, 0)),
            pl.BlockSpec((9, _EMB, _EMB), lambda s: (0, 0, 0)),
            pl.BlockSpec((1, _EMB), lambda s: (0, 0)),
            pl.BlockSpec((_BB, 16, _EMB), lambda s: (s, 0, 0)),
        ],
        scratch_shapes=[
            pltpu.VMEM((2 * _BB, 16, 16, _EMB), jnp.float32),
            pltpu.VMEM((2 * _BB, 8, 8, _EMB), jnp.float32),
        ],
        out_specs=[
            pl.BlockSpec((_BB, 16, _EMB), lambda s: (s, 0, 0)),
            pl.BlockSpec((_BB, 16, 1), lambda s: (s, 0, 0)),
            pl.BlockSpec((16, _K), lambda s: (0, 0)),
        ],
        out_shape=[
            jax.ShapeDtypeStruct((bsz, 16, _EMB), jnp.float32),
            jax.ShapeDtypeStruct((bsz, 16, 1), jnp.float32),
            jax.ShapeDtypeStruct((16, _K), jnp.float32),
        ],
    )(input_data_first, input_data_last, W_in, b_in.reshape(1, _EMB), cbt,
      w1, conv1_b.reshape(1, _EMB), w2, conv2_b.reshape(1, _EMB), rv)

    qs = q.reshape(bsz, _EMB, 16).transpose(0, 2, 1).reshape(bsz * 16, _EMB)

    out, perp, used_f = pl.pallas_call(
        _kernel_b,
        grid=(bsz * 16 // _RB,),
        in_specs=[
            pl.BlockSpec((_RB, _EMB), lambda s: (s, 0)),
            pl.BlockSpec((_EMB, _DIM), lambda s: (0, 0)),
            pl.BlockSpec((1, _DIM), lambda s: (0, 0)),
            pl.BlockSpec((16, _K), lambda s: (0, 0)),
            pl.BlockSpec((1, _K), lambda s: (0, 0)),
        ],
        out_specs=[
            pl.BlockSpec((_RB, _DIM), lambda s: (s, 0)),
            pl.BlockSpec((1, 1), lambda s: (0, 0)),
            pl.BlockSpec((1, _K), lambda s: (0, 0)),
        ],
        out_shape=[
            jax.ShapeDtypeStruct((bsz * 16, _DIM), jnp.float32),
            jax.ShapeDtypeStruct((1, 1), jnp.float32),
            jax.ShapeDtypeStruct((1, _K), jnp.float32),
        ],
    )(qs, W_out, b_out.reshape(1, _DIM), acc,
      codebooks_used.astype(jnp.float32).reshape(1, _K))

    out = out.reshape(bsz, 16, _DIM)
    perplexity = perp.reshape(())
    used = used_f.reshape(_K).astype(jnp.int32)
    min_indices = idxf.reshape(bsz, 16).astype(jnp.int32)
    return out, perplexity, used, min_indices


# speedup vs baseline: 2.0542x; 1.1171x over previous
import functools

import jax
import jax.numpy as jnp
import numpy as np
from jax.experimental import pallas as pl
from jax.experimental.pallas import tpu as pltpu

_DIM = 768
_EMB = 64
_K = 1024
_B = 128
_SEQ = 256
_EPS = 1e-12
_BB = 8
_RB = 256
_N = _B * 16

_HIGH = jax.lax.Precision.DEFAULT


def _dot(a, b):
    return jax.lax.dot_general(a, b, (((1,), (0,)), ((), ())),
                               precision=_HIGH,
                               preferred_element_type=jnp.float32)


def _tap_views(scr, j, ohw, m):
    p = {}
    for hp in (0, 1):
        for wp in (0, 1):
            v = scr[pl.ds(j, 1), pl.Slice(hp, ohw, 2), pl.Slice(wp, ohw, 2), :]
            p[(hp, wp)] = v.reshape(ohw * ohw, _EMB)
    z = jnp.zeros((ohw, _EMB), jnp.float32)
    z1 = jnp.zeros((1, _EMB), jnp.float32)

    def sh(x):
        return jnp.concatenate([z, x[:-ohw]], axis=0)

    def sw(x):
        return jnp.concatenate([z1, x[:-1]], axis=0) * m

    views = []
    for hp, shf in ((1, True), (0, False), (1, False)):
        for wp, swf in ((1, True), (0, False), (1, False)):
            v = p[(hp, wp)]
            if swf:
                v = sw(v)
            if shf:
                v = sh(v)
            views.append(v)
    return views


def _conv(scr, j, ohw, m, w_ref):
    vs = _tap_views(scr, j, ohw, m)
    h = _dot(vs[0], w_ref[0])
    for t in range(1, 9):
        h = h + _dot(vs[t], w_ref[t])
    return h


def _kernel_a(first_ref, last_ref, win_ref, bin_ref, cbt_ref, w1_ref, b1_ref,
              w2_ref, b2_ref, rv_ref, q_ref, idx_ref, acc_ref,
              scr1, scr2):
    step = pl.program_id(0)

    @pl.when(step == 0)
    def _():
        acc_ref[...] = jnp.zeros_like(acc_ref)

    win = win_ref[...]
    b_in = bin_ref[...]
    b1 = b1_ref[...]
    b2 = b2_ref[...]
    cbt = cbt_ref[...]
    cb_sq = jnp.sum(cbt * cbt, axis=0, keepdims=True)
    nr = _BB * 16
    lane = jax.lax.broadcasted_iota(jnp.int32, (nr, _K), 1).astype(jnp.float32)

    a_f = _dot(first_ref[...].reshape(_BB * _SEQ, _DIM), win) + b_in
    a_l = _dot(last_ref[...].reshape(_BB * _SEQ, _DIM), win) + b_in

    m1 = (jax.lax.broadcasted_iota(jnp.int32, (64, 1), 0) % 8
          != 0).astype(jnp.float32)
    m2 = (jax.lax.broadcasted_iota(jnp.int32, (16, 1), 0) % 4
          != 0).astype(jnp.float32)
    for i in range(_BB):
        scr1[i] = a_f[i * _SEQ:(i + 1) * _SEQ].reshape(16, 16, _EMB)
        scr1[_BB + i] = a_l[i * _SEQ:(i + 1) * _SEQ].reshape(16, 16, _EMB)
    for j in range(2 * _BB):
        r = jax.nn.relu(_conv(scr1, j, 8, m1, w1_ref) + b1)
        scr2[j] = r.reshape(8, 8, _EMB)
    xs = []
    for i in range(_BB):
        xs.append((_conv(scr2, _BB + i, 4, m2, w2_ref) + b2)
                  - (_conv(scr2, i, 4, m2, w2_ref) + b2))
    x = jnp.concatenate(xs, axis=0)

    x_sq = jnp.sum(x * x, axis=1, keepdims=True)
    dist = x_sq - 2.0 * _dot(x, cbt) + cb_sq
    md = jnp.min(dist, axis=1, keepdims=True)
    idxf = jnp.min(jnp.where(dist <= md, lane, float(_K)), axis=1,
                   keepdims=True)

    n_res = jnp.sqrt(jnp.maximum(md, 0.0))
    rv = rv_ref[...].reshape(nr, _EMB)
    n_rv = jnp.sqrt(jnp.sum(rv * rv, axis=1, keepdims=True))
    q_ref[...] = (x + (n_res / n_rv + _EPS) * rv).reshape(_BB, 16, _EMB)
    idx_ref[...] = idxf.reshape(_BB, 16, 1)

    onehot = jnp.where(lane == idxf, 1.0, 0.0)
    acc = acc_ref[...]
    for i in range(_BB):
        acc = acc + onehot[i * 16:(i + 1) * 16]
    acc_ref[...] = acc


def _kernel_b(qs_ref, wout_ref, bout_ref, acc_ref, cu_ref,
              out_ref, perp_ref, used_ref):
    @pl.when(pl.program_id(0) == 0)
    def _():
        counts = jnp.sum(acc_ref[...], axis=0, keepdims=True)
        used_ref[...] = cu_ref[...] + counts
        avg = counts * (1.0 / _N)
        ent = jnp.sum(avg * jnp.log(avg + _EPS), axis=1, keepdims=True)
        perp_ref[...] = jnp.exp(-ent)

    out_ref[...] = _dot(qs_ref[...], wout_ref[...]) + bout_ref[...]


def kernel(input_data_first, input_data_last, codebooks, W_in, b_in,
           conv1_w, conv1_b, conv2_w, conv2_b, W_out, b_out, codebooks_used):
    bsz = input_data_first.shape[0]
    cbt = codebooks.T
    w1 = conv1_w.transpose(2, 3, 1, 0).reshape(9, _EMB, _EMB)
    w2 = conv2_w.transpose(2, 3, 1, 0).reshape(9, _EMB, _EMB)
    rv = jax.random.normal(jax.random.key(42), (_N, _EMB),
                           jnp.float32).reshape(bsz, 16, _EMB)
    steps = bsz // _BB
    q, idxf, acc = pl.pallas_call(
        _kernel_a,
        grid=(steps,),
        in_specs=[
            pl.BlockSpec((_BB, _SEQ, _DIM), lambda s: (s, 0, 0)),
            pl.BlockSpec((_BB, _SEQ, _DIM), lambda s: (s, 0, 0)),
            pl.BlockSpec((_DIM, _EMB), lambda s: (0, 0)),
            pl.BlockSpec((1, _EMB), lambda s: (0, 0)),
            pl.BlockSpec((_EMB, _K), lambda s: (0, 0)),
            pl.BlockSpec((9, _EMB, _EMB), lambda s: (0, 0, 0)),
            pl.BlockSpec((1, _EMB), lambda s: (0, 0)),
            pl.BlockSpec((9, _EMB, _EMB), lambda s: (0, 0, 0)),
            pl.BlockSpec((1, _EMB), lambda s: (0, 0)),
            pl.BlockSpec((_BB, 16, _EMB), lambda s: (s, 0, 0)),
        ],
        scratch_shapes=[
            pltpu.VMEM((2 * _BB, 16, 16, _EMB), jnp.float32),
            pltpu.VMEM((2 * _BB, 8, 8, _EMB), jnp.float32),
        ],
        out_specs=[
            pl.BlockSpec((_BB, 16, _EMB), lambda s: (s, 0, 0)),
            pl.BlockSpec((_BB, 16, 1), lambda s: (s, 0, 0)),
            pl.BlockSpec((16, _K), lambda s: (0, 0)),
        ],
        out_shape=[
            jax.ShapeDtypeStruct((bsz, 16, _EMB), jnp.float32),
            jax.ShapeDtypeStruct((bsz, 16, 1), jnp.float32),
            jax.ShapeDtypeStruct((16, _K), jnp.float32),
        ],
    )(input_data_first, input_data_last, W_in, b_in.reshape(1, _EMB), cbt,
      w1, conv1_b.reshape(1, _EMB), w2, conv2_b.reshape(1, _EMB), rv)

    qs = q.reshape(bsz, _EMB, 16).transpose(0, 2, 1).reshape(bsz * 16, _EMB)

    out, perp, used_f = pl.pallas_call(
        _kernel_b,
        grid=(bsz * 16 // _RB,),
        in_specs=[
            pl.BlockSpec((_RB, _EMB), lambda s: (s, 0)),
            pl.BlockSpec((_EMB, _DIM), lambda s: (0, 0)),
            pl.BlockSpec((1, _DIM), lambda s: (0, 0)),
            pl.BlockSpec((16, _K), lambda s: (0, 0)),
            pl.BlockSpec((1, _K), lambda s: (0, 0)),
        ],
        out_specs=[
            pl.BlockSpec((_RB, _DIM), lambda s: (s, 0)),
            pl.BlockSpec((1, 1), lambda s: (0, 0)),
            pl.BlockSpec((1, _K), lambda s: (0, 0)),
        ],
        out_shape=[
            jax.ShapeDtypeStruct((bsz * 16, _DIM), jnp.float32),
            jax.ShapeDtypeStruct((1, 1), jnp.float32),
            jax.ShapeDtypeStruct((1, _K), jnp.float32),
        ],
    )(qs, W_out, b_out.reshape(1, _DIM), acc,
      codebooks_used.astype(jnp.float32).reshape(1, _K))

    out = out.reshape(bsz, 16, _DIM)
    perplexity = perp.reshape(())
    used = used_f.reshape(_K).astype(jnp.int32)
    min_indices = idxf.reshape(bsz, 16).astype(jnp.int32)
    return out, perplexity, used, min_indices
